# baseline (speedup 1.0000x reference)
import functools

import jax
import jax.numpy as jnp
from jax import lax
from jax.experimental import pallas as pl
from jax.experimental.pallas import tpu as pltpu
from jax.experimental.pallas import tpu_sc as plsc

_IN = (64, 96, 512)
_C0, _C1 = 128, 300
_W = _C1 - _C0
_R = _IN[0] * _IN[1]
_N = _R * _W
_K = int(0.1 * _N)
_EPS_LOG = 0.001

_NC, _NS = 2, 16
_NT = _NC * _NS
_RPT = _R // _NT
_CHROWS = 48
_NCH = _RPT // _CHROWS
_CHW = _CHROWS * _IN[2]
_NB = 1 << 16
_I32_MIN = -(2 ** 31)

_ROW_BLK = 512
_G = _R // _ROW_BLK


def _sc_round_body(lo_round, x_hbm, p_hbm, hist_out, dbuf, hist, pbuf):
    wid = lax.axis_index("s") * _NC + lax.axis_index("c")
    ones = jnp.full((16,), 1, jnp.int32)
    lane = lax.iota(jnp.int32, 16)
    colmask = lane < (_W - 10 * 16)
    allmask = lane < 16

    if lo_round:
        pltpu.sync_copy(p_hbm.at[0], pbuf)
        pvec = pbuf[pl.ds(0, 16)]

    @plsc.parallel_loop(0, _NB // 16, 1, unroll=8)
    def _zero(i):
        hist[pl.ds(i * 16, 16)] = jnp.zeros((16,), jnp.int32)

    def row_body(r):
        base = r * _IN[2] + _C0
        for c in range(11):
            bits = dbuf[pl.ds(base + c * 16, 16)]
            u = jnp.where(bits < 0, ~bits, bits ^ _I32_MIN)
            if lo_round:
                idx = u & 0xFFFF
                m = lax.shift_right_logical(u, 16) == pvec
                if c == 10:
                    m = m & colmask
                plsc.addupdate_scatter(hist, [idx], ones, mask=m)
            else:
                idx = lax.shift_right_logical(u, 16)
                m = colmask if c == 10 else allmask
                plsc.addupdate_scatter(hist, [idx], ones, mask=m)

    for ch in range(_NCH):
        off = (wid * _RPT + ch * _CHROWS) * _IN[2]
        pltpu.sync_copy(x_hbm.at[pl.ds(off, _CHW)], dbuf)
        plsc.parallel_loop(0, _CHROWS, 1, unroll=4)(row_body)

    pltpu.sync_copy(hist, hist_out.at[wid])


def _make_sc_round(lo_round):
    mesh = plsc.VectorSubcoreMesh(core_axis_name="c", subcore_axis_name="s")
    scratch = [
        pltpu.VMEM((_CHW,), jnp.int32),
        pltpu.VMEM((_NB,), jnp.int32),
        pltpu.VMEM((128,), jnp.int32),
    ]
    return functools.partial(
        pl.kernel,
        out_type=jax.ShapeDtypeStruct((_NT, _NB), jnp.int32),
        mesh=mesh,
        scratch_types=scratch,
        compiler_params=pltpu.CompilerParams(needs_layout_passes=False),
    )(functools.partial(_sc_round_body, lo_round))


_sc_hi = _make_sc_round(False)
_sc_lo = _make_sc_round(True)


def _merge_hist(h_ref):
    h = h_ref[0]
    for w in range(1, _NT):
        h = h + h_ref[w]
    return h


def _search16(H, flat, k):
    lo = jnp.int32(0)
    for j in range(16):
        mid = lo | (1 << (15 - j))
        c = jnp.sum(jnp.where(flat < mid, H, 0))
        lo = jnp.where(c <= k, mid, lo)
    below = jnp.sum(jnp.where(flat < lo, H, 0))
    return lo, below


def _flat_iota():
    r = lax.broadcasted_iota(jnp.int32, (_NB // 128, 128), 0)
    c = lax.broadcasted_iota(jnp.int32, (_NB // 128, 128), 1)
    return r * 128 + c


def _resolve_hi_body(h_ref, o_ref):
    H = _merge_hist(h_ref)
    p, below = _search16(H, _flat_iota(), jnp.int32(_K))
    krem = jnp.int32(_K) - below
    row = lax.broadcasted_iota(jnp.int32, (8, 128), 0)
    o_ref[...] = jnp.where(row == 0, p, krem)


def _resolve_lo_body(h_ref, pk_ref, ml_ref, o_ref):
    H = _merge_hist(h_ref)
    p = pk_ref[0, 0]
    krem = pk_ref[1, 0]
    lo, _ = _search16(H, _flat_iota(), krem)
    u_k = lax.shift_left(p, 16) | lo
    uv = u_k + jnp.zeros((8, 128), jnp.int32)
    v = uv ^ jnp.int32(_I32_MIN)
    fb = v ^ (lax.shift_right_arithmetic(v, 31) & jnp.int32(0x7FFFFFFF))
    eps = lax.bitcast_convert_type(fb, jnp.float32)
    ylo = jnp.log10(jnp.maximum(eps, jnp.float32(_EPS_LOG)))
    yhi = jnp.maximum(ml_ref[0], ylo)
    inv = 1.0 / (yhi - ylo)
    row = lax.broadcasted_iota(jnp.int32, (8, 128), 0)
    o_ref[...] = jnp.where(row == 0, ylo, inv)


def _log_body(x_ref, l_ref, ml_ref):
    g = pl.program_id(0)
    L = jnp.log10(jnp.maximum(x_ref[:, _C0:_C1], jnp.float32(_EPS_LOG)))
    l_ref[...] = L

    @pl.when(g == 0)
    def _():
        ml_ref[0] = jnp.float32(-jnp.inf)

    ml_ref[0] = jnp.maximum(ml_ref[0], jnp.max(L))


def _norm_body(l_ref, s_ref, o_ref):
    ylo = s_ref[0, 0]
    inv = s_ref[1, 0]
    o_ref[...] = (jnp.maximum(l_ref[...], ylo) - ylo) * inv


def kernel(x):
    xf = lax.bitcast_convert_type(x, jnp.int32).reshape(-1)
    x2 = x.reshape(_R, _IN[2])

    hist1 = _sc_hi(xf, jnp.zeros((8, 128), jnp.int32))
    L, maxL = pl.pallas_call(
        _log_body,
        grid=(_G,),
        in_specs=[pl.BlockSpec((_ROW_BLK, _IN[2]), lambda g: (g, 0))],
        out_specs=[
            pl.BlockSpec((_ROW_BLK, _W), lambda g: (g, 0)),
            pl.BlockSpec(memory_space=pltpu.SMEM),
        ],
        out_shape=[
            jax.ShapeDtypeStruct((_R, _W), jnp.float32),
            jax.ShapeDtypeStruct((1,), jnp.float32),
        ],
    )(x2)

    pk = pl.pallas_call(
        _resolve_hi_body,
        out_shape=jax.ShapeDtypeStruct((8, 128), jnp.int32),
    )(hist1.reshape(_NT, _NB // 128, 128))

    hist2 = _sc_lo(xf, pk)

    scal = pl.pallas_call(
        _resolve_lo_body,
        in_specs=[
            pl.BlockSpec(memory_space=pltpu.VMEM),
            pl.BlockSpec(memory_space=pltpu.SMEM),
            pl.BlockSpec(memory_space=pltpu.SMEM),
        ],
        out_shape=jax.ShapeDtypeStruct((8, 128), jnp.float32),
    )(hist2.reshape(_NT, _NB // 128, 128), pk, maxL)

    out = pl.pallas_call(
        _norm_body,
        grid=(_G,),
        in_specs=[
            pl.BlockSpec((_ROW_BLK, _W), lambda g: (g, 0)),
            pl.BlockSpec(memory_space=pltpu.SMEM),
        ],
        out_specs=pl.BlockSpec((_ROW_BLK, _W), lambda g: (g, 0)),
        out_shape=jax.ShapeDtypeStruct((_R, _W), jnp.float32),
    )(L, scal)
    return out.reshape(_IN[0], _IN[1], _W)


# speedup vs baseline: 2.3327x; 2.3327x over previous
import jax
import jax.numpy as jnp
from jax import lax
from jax.experimental import pallas as pl
from jax.experimental.pallas import tpu as pltpu

_IN = (64, 96, 512)
_C0, _C1 = 128, 300
_W = _C1 - _C0
_R = _IN[0] * _IN[1]
_N = _R * _W
_K = int(0.1 * _N)
_EPS_LOG = 0.001
_I32_MIN = -(2 ** 31)

_BLK = 512
_G = _R // _BLK


def _s32(val):
    val &= 0xFFFFFFFF
    return val - (1 << 32) if val >= (1 << 31) else val


def _body(x_ref, o_ref, v_buf, c_ref):
    g = pl.program_id(0)

    @pl.when(g == 0)
    def _():
        for t in range(16):
            c_ref[t] = 0

    xs = x_ref[:, _C0:_C1]
    bits = lax.bitcast_convert_type(xs, jnp.int32)
    v = bits ^ (lax.shift_right_arithmetic(bits, 31) & jnp.int32(0x7FFFFFFF))
    v_buf[pl.ds(g * _BLK, _BLK), :] = v

    for t in range(1, 16):
        ts = jnp.int32(_s32((t << 28) ^ (1 << 31)))
        c_ref[t] = c_ref[t] + jnp.sum((v < ts).astype(jnp.int32))

    @pl.when(g == _G - 1)
    def _():
        vb = v_buf[...]

        lo = jnp.int32(0)
        for t in range(1, 16):
            lo = jnp.where(c_ref[t] <= _K, jnp.int32(_s32(t << 28)), lo)

        def step(i, lo):
            mid = lo | lax.shift_left(jnp.int32(1), 31 - i)
            c = jnp.sum((vb < (mid ^ jnp.int32(_I32_MIN))).astype(jnp.int32))
            return jnp.where(c <= _K, mid, lo)

        lo = lax.fori_loop(4, 32, step, lo)
        vk = lo ^ jnp.int32(_I32_MIN)
        fb = vk ^ (lax.shift_right_arithmetic(vk, 31) & jnp.int32(0x7FFFFFFF))
        eps = lax.bitcast_convert_type(fb, jnp.float32)

        fbs = vb ^ (lax.shift_right_arithmetic(vb, 31) & jnp.int32(0x7FFFFFFF))
        xsr = lax.bitcast_convert_type(fbs, jnp.float32)
        m = jnp.maximum(eps, jnp.float32(_EPS_LOG))
        xmax = jnp.max(xsr)
        ylo = jnp.log10(m)
        yhi = jnp.log10(jnp.maximum(xmax, m))
        o_ref[...] = (jnp.log10(jnp.maximum(xsr, m)) - ylo) / (yhi - ylo)


def kernel(x):
    x2 = x.reshape(_R, _IN[2])
    out = pl.pallas_call(
        _body,
        grid=(_G,),
        in_specs=[pl.BlockSpec((_BLK, _IN[2]), lambda g: (g, 0))],
        out_specs=pl.BlockSpec((_R, _W), lambda g: (0, 0)),
        out_shape=jax.ShapeDtypeStruct((_R, _W), jnp.float32),
        scratch_shapes=[
            pltpu.VMEM((_R, _W), jnp.int32),
            pltpu.SMEM((16,), jnp.int32),
        ],
    )(x2)
    return out.reshape(_IN[0], _IN[1], _W)
